# baseline (device time: 26967 ns/iter reference)
import jax
import jax.numpy as jnp
from jax import lax
from jax.experimental import pallas as pl
from jax.experimental.pallas import tpu as pltpu

N_DEV = 4
N_EXPERTS = 16
N_LOCAL_E = 4
K_CAP = 128


def _tdot(a, b):
    return lax.dot_general(a, b, (((0,), (0,)), ((), ())),
                           preferred_element_type=jnp.float32)


def kernel(x, router_W, route_idx, expert_W, shared_W):
    n, d = x.shape
    _, _, h = expert_W.shape
    chunk = n // N_DEV

    def body(x_ref, router_ref, idx_ref, expW_ref, sharedW_hbm,
             out_ref, coeff_ref, sharedw_vmem, y_send, y_recv,
             send_y_sems, recv_y_sems, sload_sem):
        my_pos = lax.axis_index("i")

        s_copy = pltpu.make_async_copy(sharedW_hbm, sharedw_vmem, sload_sem)
        s_copy.start()

        barrier_sem = pltpu.get_barrier_semaphore()
        for dd in range(1, N_DEV):
            pl.semaphore_signal(
                barrier_sem, inc=1,
                device_id=(lax.rem(my_pos + dd, N_DEV),),
                device_id_type=pl.DeviceIdType.MESH,
            )

        xv = x_ref[:, :]
        scores = jnp.dot(xv, router_ref[:, :],
                         preferred_element_type=jnp.float32,
                         precision=lax.Precision.HIGHEST)
        scores = scores - jnp.max(scores, axis=-1, keepdims=True)
        ex = jnp.exp(scores)
        probs = ex / jnp.sum(ex, axis=-1, keepdims=True)
        lanes = lax.broadcasted_iota(jnp.int32, (n, N_EXPERTS), 1)
        gate = probs * (lanes == idx_ref[:, :]).astype(jnp.float32)
        r16 = lax.broadcasted_iota(jnp.int32, (N_EXPERTS, N_LOCAL_E), 0)
        c4 = lax.broadcasted_iota(jnp.int32, (N_EXPERTS, N_LOCAL_E), 1)
        sel = (r16 == N_LOCAL_E * my_pos + c4).astype(jnp.float32)
        coeff_ref[:, :] = jnp.dot(gate, sel, preferred_element_type=jnp.float32)

        tri_r = lax.broadcasted_iota(jnp.int32, (chunk, chunk), 0)
        tri_c = lax.broadcasted_iota(jnp.int32, (chunk, chunk), 1)
        t_strict = (tri_c < tri_r).astype(jnp.float32)
        klane = lax.broadcasted_iota(jnp.int32, (chunk, K_CAP), 1)

        def compact(c, owner):
            idx_c = idx_ref[pl.ds(c * chunk, chunk), :]
            e_lo = N_LOCAL_E * owner
            maskf = ((idx_c >= e_lo) & (idx_c < e_lo + N_LOCAL_E)).astype(
                jnp.float32)
            rank = jnp.dot(t_strict, maskf,
                           preferred_element_type=jnp.float32)
            rank_i = rank.astype(jnp.int32)
            e_mat = (klane == rank_i).astype(jnp.float32) * maskf
            return e_mat

        def gather(c):
            e_mat = compact(c, my_pos)
            xg = _tdot(e_mat, x_ref[pl.ds(c * chunk, chunk), :])
            cfg = _tdot(e_mat, coeff_ref[pl.ds(c * chunk, chunk), :])
            return e_mat, xg, cfg

        def pair_y(ga, gb):
            xg2 = jnp.concatenate([ga[1], gb[1]], axis=0)
            cfg2 = jnp.concatenate([ga[2], gb[2]], axis=0)
            acc = jnp.zeros((2 * K_CAP, h), jnp.float32)
            for el in range(N_LOCAL_E):
                y = jnp.dot(xg2, expW_ref[el],
                            preferred_element_type=jnp.float32)
                acc = acc + cfg2[:, el:el + 1] * y
            return acc[:K_CAP, :], acc[K_CAP:, :]

        def send(dd, y_val):
            slot = dd - 1
            tgt = lax.rem(my_pos + dd, N_DEV)
            y_send[slot, :, :] = y_val.astype(jnp.bfloat16)
            rdma = pltpu.make_async_remote_copy(
                src_ref=y_send.at[slot], dst_ref=y_recv.at[slot],
                send_sem=send_y_sems.at[slot], recv_sem=recv_y_sems.at[slot],
                device_id=(tgt,), device_id_type=pl.DeviceIdType.MESH,
            )
            rdma.start()
            return rdma

        rdmas = {}
        g2 = gather(lax.rem(my_pos + 2, N_DEV))
        g1 = gather(lax.rem(my_pos + 1, N_DEV))
        y2, y1 = pair_y(g2, g1)
        pl.semaphore_wait(barrier_sem, N_DEV - 1)
        rdmas[2] = send(2, y2)
        rdmas[1] = send(1, y1)

        g3 = gather(lax.rem(my_pos + 3, N_DEV))
        g0 = gather(my_pos)
        y3, y_own = pair_y(g3, g0)
        rdmas[3] = send(3, y3)
        own = jnp.dot(g0[0], y_own, preferred_element_type=jnp.float32)
        s_copy.wait()
        shared_chunk = jnp.dot(
            x_ref[pl.ds(my_pos * chunk, chunk), :], sharedw_vmem[:, :],
            preferred_element_type=jnp.float32,
        )
        acc = own + shared_chunk

        e_srcs = {}
        for dd in (2, 1, 3):
            src = lax.rem(my_pos + N_DEV - dd, N_DEV)
            e_srcs[dd] = compact(my_pos, src)

        for dd in (2, 1, 3):
            rdmas[dd].wait()
            slot = dd - 1
            acc = acc + jnp.dot(
                e_srcs[dd], y_recv[slot, :, :],
                preferred_element_type=jnp.float32,
            )
        out_ref[:, :] = acc

    return pl.pallas_call(
        body,
        out_shape=jax.ShapeDtypeStruct((chunk, h), jnp.float32),
        in_specs=[
            pl.BlockSpec(memory_space=pltpu.VMEM),
            pl.BlockSpec(memory_space=pltpu.VMEM),
            pl.BlockSpec(memory_space=pltpu.VMEM),
            pl.BlockSpec(memory_space=pltpu.VMEM),
            pl.BlockSpec(memory_space=pltpu.MemorySpace.HBM),
        ],
        out_specs=pl.BlockSpec(memory_space=pltpu.VMEM),
        scratch_shapes=[
            pltpu.VMEM((n, N_LOCAL_E), jnp.float32),
            pltpu.VMEM((d, h), jnp.float32),
            pltpu.VMEM((3, K_CAP, h), jnp.bfloat16),
            pltpu.VMEM((3, K_CAP, h), jnp.bfloat16),
            pltpu.SemaphoreType.DMA((3,)),
            pltpu.SemaphoreType.DMA((3,)),
            pltpu.SemaphoreType.DMA,
        ],
        compiler_params=pltpu.CompilerParams(collective_id=0),
    )(x, router_W, route_idx, expert_W, shared_W)


# device time: 25401 ns/iter; 1.0617x vs baseline; 1.0617x over previous
import jax
import jax.numpy as jnp
from jax import lax
from jax.experimental import pallas as pl
from jax.experimental.pallas import tpu as pltpu

N_DEV = 4
N_EXPERTS = 16
N_LOCAL_E = 4
K_CAP = 128


def _tdot(a, b):
    return lax.dot_general(a, b, (((0,), (0,)), ((), ())),
                           preferred_element_type=jnp.float32)


def kernel(x, router_W, route_idx, expert_W, shared_W):
    n, d = x.shape
    _, _, h = expert_W.shape
    chunk = n // N_DEV

    def body(x_ref, router_ref, idx_ref, expW_ref, sharedW_hbm,
             out_ref, coeff_ref, sharedw_vmem, y_send, y_recv,
             send_y_sems, recv_y_sems, sload_sem):
        my_pos = lax.axis_index("i")

        s_copy = pltpu.make_async_copy(sharedW_hbm, sharedw_vmem, sload_sem)
        s_copy.start()

        barrier_sem = pltpu.get_barrier_semaphore()
        for dd in range(1, N_DEV):
            pl.semaphore_signal(
                barrier_sem, inc=1,
                device_id=(lax.rem(my_pos + dd, N_DEV),),
                device_id_type=pl.DeviceIdType.MESH,
            )

        xv = x_ref[:, :]
        scores = jnp.dot(xv, router_ref[:, :],
                         preferred_element_type=jnp.float32,
                         precision=lax.Precision.HIGHEST)
        scores = scores - jnp.max(scores, axis=-1, keepdims=True)
        ex = jnp.exp(scores)
        probs = ex / jnp.sum(ex, axis=-1, keepdims=True)
        lanes = lax.broadcasted_iota(jnp.int32, (n, N_EXPERTS), 1)
        gate = probs * (lanes == idx_ref[:, :]).astype(jnp.float32)
        r16 = lax.broadcasted_iota(jnp.int32, (N_EXPERTS, N_LOCAL_E), 0)
        c4 = lax.broadcasted_iota(jnp.int32, (N_EXPERTS, N_LOCAL_E), 1)
        sel = (r16 == N_LOCAL_E * my_pos + c4).astype(jnp.float32)
        coeff_ref[:, :] = jnp.dot(gate, sel, preferred_element_type=jnp.float32)

        tri_r = lax.broadcasted_iota(jnp.int32, (chunk, chunk), 0)
        tri_c = lax.broadcasted_iota(jnp.int32, (chunk, chunk), 1)
        t_strict = (tri_c < tri_r).astype(jnp.float32)
        klane = lax.broadcasted_iota(jnp.int32, (chunk, K_CAP), 1)

        def compact(c, owner):
            idx_c = idx_ref[pl.ds(c * chunk, chunk), :]
            e_lo = N_LOCAL_E * owner
            maskf = ((idx_c >= e_lo) & (idx_c < e_lo + N_LOCAL_E)).astype(
                jnp.float32)
            rank = jnp.dot(t_strict, maskf,
                           preferred_element_type=jnp.float32)
            rank_i = rank.astype(jnp.int32)
            e_mat = (klane == rank_i).astype(jnp.float32) * maskf
            return e_mat

        def sparse_y(c, e_mat):
            xg = _tdot(e_mat, x_ref[pl.ds(c * chunk, chunk), :])
            cfg = _tdot(e_mat, coeff_ref[pl.ds(c * chunk, chunk), :])
            acc = jnp.zeros((K_CAP, h), jnp.float32)
            for el in range(N_LOCAL_E):
                y = jnp.dot(xg, expW_ref[el], preferred_element_type=jnp.float32)
                acc = acc + cfg[:, el:el + 1] * y
            return acc

        rdmas = {}
        for dd in (2, 1, 3):
            slot = dd - 1
            tgt = lax.rem(my_pos + dd, N_DEV)
            c = tgt
            e_mat = compact(c, my_pos)
            y_send[slot, :, :] = sparse_y(c, e_mat).astype(jnp.bfloat16)
            if dd == 2:
                pl.semaphore_wait(barrier_sem, N_DEV - 1)
            rdma_y = pltpu.make_async_remote_copy(
                src_ref=y_send.at[slot], dst_ref=y_recv.at[slot],
                send_sem=send_y_sems.at[slot], recv_sem=recv_y_sems.at[slot],
                device_id=(tgt,), device_id_type=pl.DeviceIdType.MESH,
            )
            rdma_y.start()
            rdmas[dd] = rdma_y

        e_own = compact(my_pos, my_pos)
        y_own = sparse_y(my_pos, e_own)
        own = jnp.dot(e_own, y_own, preferred_element_type=jnp.float32)
        s_copy.wait()
        shared_chunk = jnp.dot(
            x_ref[pl.ds(my_pos * chunk, chunk), :], sharedw_vmem[:, :],
            preferred_element_type=jnp.float32,
        )
        acc = own + shared_chunk

        e_srcs = {}
        for dd in (2, 1, 3):
            src = lax.rem(my_pos + N_DEV - dd, N_DEV)
            e_srcs[dd] = compact(my_pos, src)

        for dd in (2, 1, 3):
            rdmas[dd].wait()
            slot = dd - 1
            acc = acc + jnp.dot(
                e_srcs[dd], y_recv[slot, :, :],
                preferred_element_type=jnp.float32,
            )
        out_ref[:, :] = acc

    return pl.pallas_call(
        body,
        out_shape=jax.ShapeDtypeStruct((chunk, h), jnp.float32),
        in_specs=[
            pl.BlockSpec(memory_space=pltpu.VMEM),
            pl.BlockSpec(memory_space=pltpu.VMEM),
            pl.BlockSpec(memory_space=pltpu.VMEM),
            pl.BlockSpec(memory_space=pltpu.VMEM),
            pl.BlockSpec(memory_space=pltpu.MemorySpace.HBM),
        ],
        out_specs=pl.BlockSpec(memory_space=pltpu.VMEM),
        scratch_shapes=[
            pltpu.VMEM((n, N_LOCAL_E), jnp.float32),
            pltpu.VMEM((d, h), jnp.float32),
            pltpu.VMEM((3, K_CAP, h), jnp.bfloat16),
            pltpu.VMEM((3, K_CAP, h), jnp.bfloat16),
            pltpu.SemaphoreType.DMA((3,)),
            pltpu.SemaphoreType.DMA((3,)),
            pltpu.SemaphoreType.DMA,
        ],
        compiler_params=pltpu.CompilerParams(collective_id=0),
    )(x, router_W, route_idx, expert_W, shared_W)
